# baseline (speedup 1.0000x reference)
import functools

import jax
import jax.numpy as jnp
from jax import lax
from jax.experimental import pallas as pl
from jax.experimental.pallas import tpu as pltpu
from jax.experimental.pallas import tpu_sc as plsc

EMBED = 64
LANES = 16
NC, NS = 2, 16
NW = NC * NS
BATCH = 4096
SEQ = 200
TOTAL = BATCH * SEQ
BPW = BATCH // NW
PER_W = BPW * SEQ
ER = EMBED // 8
TILE = 8 * 128


@functools.partial(
    pl.kernel,
    mesh=plsc.VectorSubcoreMesh(core_axis_name="c", subcore_axis_name="s"),
    out_type=jax.ShapeDtypeStruct((SEQ, ER, NW, TILE), jnp.float32),
    scratch_types=[
        pltpu.VMEM((PER_W,), jnp.int32),
        pltpu.VMEM((PER_W,), jnp.int32),
        pltpu.VMEM((2, BPW, EMBED), jnp.float32),
        pltpu.VMEM((2, ER, TILE), jnp.float32),
        pltpu.SemaphoreType.DMA,
        pltpu.SemaphoreType.DMA,
        pltpu.SemaphoreType.DMA,
        pltpu.SemaphoreType.DMA,
    ],
    compiler_params=pltpu.CompilerParams(
        use_tc_tiling_on_sc=False, needs_layout_passes=False),
)
def _sc_gather(table_hbm, idx_hbm, out_hbm, idx_v, idxt_v, rows_v, tiles_v,
               g0, g1, o0, o1):
    gsem = (g0, g1)
    osem = (o0, o1)
    wid = lax.axis_index("s") * NC + lax.axis_index("c")
    base = wid * PER_W
    pltpu.sync_copy(idx_hbm.at[pl.ds(base, PER_W)], idx_v)

    iota = lax.iota(jnp.int32, LANES)
    pre_s = [(16 * g + iota) * SEQ for g in range(BPW // LANES)]

    @plsc.parallel_loop(0, SEQ, unroll=2)
    def _build_idxt(s):
        for g in range(BPW // LANES):
            val = plsc.load_gather(idx_v, [pre_s[g] + s])
            idxt_v[pl.ds(s * BPW + 16 * g, LANES)] = val

    def fire_g(s, b):
        pltpu.async_copy(
            table_hbm.at[idxt_v.at[pl.ds(s * BPW, BPW)]],
            rows_v.at[b],
            gsem[b],
        )

    def wait_g(b):
        pltpu.make_async_copy(
            table_hbm.at[pl.ds(0, BPW), :],
            rows_v.at[b],
            gsem[b],
        ).wait()

    er_vec = [(16 * ev + iota) >> 3 for ev in range(EMBED // LANES)]
    colbase = [((16 * ev + iota) & 7) * 128 for ev in range(EMBED // LANES)]

    def transpose(b):
        @plsc.parallel_loop(0, BPW, unroll=8)
        def _t(bl):
            for ev in range(EMBED // LANES):
                val = rows_v[b, bl, pl.ds(16 * ev, LANES)]
                plsc.store_scatter(
                    tiles_v.at[b], [er_vec[ev], colbase[ev] + bl], val)

    def fire_o(s, b):
        pltpu.async_copy(
            tiles_v.at[b],
            out_hbm.at[s, :, wid],
            osem[b],
        )

    def wait_o(b):
        pltpu.make_async_copy(
            tiles_v.at[b],
            out_hbm.at[0, :, 0],
            osem[b],
        ).wait()

    fire_g(0, 0)
    fire_g(1, 1)
    for u in (0, 1):
        wait_g(u)
        transpose(u)
        fire_o(u, u)
        fire_g(u + 2, u)

    def unit(u, b):
        wait_g(b)
        wait_o(b)
        transpose(b)
        fire_o(u, b)
        fire_g(u + 2, b)

    def pair(p, carry):
        u = 2 + 2 * p
        unit(u, 0)
        unit(u + 1, 1)
        return carry

    lax.fori_loop(0, (SEQ - 4) // 2, pair, 0)
    for u in (SEQ - 2, SEQ - 1):
        b = u % 2
        wait_g(b)
        wait_o(b)
        transpose(b)
        fire_o(u, b)
    for b in (0, 1):
        wait_o(b)


def kernel(tokens, wte_weight):
    idx1d = tokens.astype(jnp.int32).reshape(TOTAL)
    out4 = _sc_gather(wte_weight, idx1d)
    o5 = out4.reshape(SEQ, ER, NW, 8, 128)
    return o5.transpose(2, 4, 0, 1, 3).reshape(BATCH, SEQ, EMBED)


# speedup vs baseline: 1.0072x; 1.0072x over previous
import functools

import jax
import jax.numpy as jnp
from jax import lax
from jax.experimental import pallas as pl
from jax.experimental.pallas import tpu as pltpu
from jax.experimental.pallas import tpu_sc as plsc

EMBED = 64
LANES = 16
NC, NS = 2, 16
NW = NC * NS
BATCH = 4096
SEQ = 200
TOTAL = BATCH * SEQ
BPW = BATCH // NW
PER_W = BPW * SEQ
ER = EMBED // 8
TILE = 8 * 128


@functools.partial(
    pl.kernel,
    mesh=plsc.VectorSubcoreMesh(core_axis_name="c", subcore_axis_name="s"),
    out_type=jax.ShapeDtypeStruct((SEQ, ER, NW, TILE), jnp.float32),
    scratch_types=[
        pltpu.VMEM((PER_W,), jnp.int32),
        pltpu.VMEM((PER_W,), jnp.int32),
        pltpu.VMEM((2, BPW, EMBED), jnp.float32),
        pltpu.VMEM((2, ER, TILE), jnp.float32),
        pltpu.SemaphoreType.DMA,
        pltpu.SemaphoreType.DMA,
        pltpu.SemaphoreType.DMA,
        pltpu.SemaphoreType.DMA,
    ],
    compiler_params=pltpu.CompilerParams(
        use_tc_tiling_on_sc=False, needs_layout_passes=False),
)
def _sc_gather(table_hbm, idx_hbm, out_hbm, idx_v, idxt_v, rows_v, tiles_v,
               g0, g1, o0, o1):
    gsem = (g0, g1)
    osem = (o0, o1)
    wid = lax.axis_index("s") * NC + lax.axis_index("c")
    base = wid * PER_W
    pltpu.sync_copy(idx_hbm.at[pl.ds(base, PER_W)], idx_v)

    iota = lax.iota(jnp.int32, LANES)
    pre_s = [(16 * g + iota) * SEQ for g in range(BPW // LANES)]

    @plsc.parallel_loop(0, SEQ, unroll=2)
    def _build_idxt(s):
        for g in range(BPW // LANES):
            val = plsc.load_gather(idx_v, [pre_s[g] + s])
            idxt_v[pl.ds(s * BPW + 16 * g, LANES)] = val

    def fire_g(s, b):
        pltpu.async_copy(
            table_hbm.at[idxt_v.at[pl.ds(s * BPW, BPW)]],
            rows_v.at[b],
            gsem[b],
        )

    def wait_g(b):
        pltpu.make_async_copy(
            table_hbm.at[pl.ds(0, BPW), :],
            rows_v.at[b],
            gsem[b],
        ).wait()

    row_vec = [16 * g + iota for g in range(BPW // LANES)]

    def transpose(b):
        @plsc.parallel_loop(0, ER, unroll=4)
        def _t(eg):
            for eo in range(8):
                col = jnp.full((LANES,), eg * 8 + eo, jnp.int32)
                for g in range(BPW // LANES):
                    val = plsc.load_gather(rows_v.at[b], [row_vec[g], col])
                    tiles_v[b, eg, pl.ds(eo * 128 + 16 * g, LANES)] = val

    def fire_o(s, b):
        pltpu.async_copy(
            tiles_v.at[b],
            out_hbm.at[s, :, wid],
            osem[b],
        )

    def wait_o(b):
        pltpu.make_async_copy(
            tiles_v.at[b],
            out_hbm.at[0, :, 0],
            osem[b],
        ).wait()

    fire_g(0, 0)
    fire_g(1, 1)
    for u in (0, 1):
        wait_g(u)
        transpose(u)
        fire_o(u, u)
        fire_g(u + 2, u)

    def unit(u, b):
        wait_g(b)
        wait_o(b)
        transpose(b)
        fire_o(u, b)
        fire_g(u + 2, b)

    def pair(p, carry):
        u = 2 + 2 * p
        unit(u, 0)
        unit(u + 1, 1)
        return carry

    lax.fori_loop(0, (SEQ - 4) // 2, pair, 0)
    for u in (SEQ - 2, SEQ - 1):
        b = u % 2
        wait_g(b)
        wait_o(b)
        transpose(b)
        fire_o(u, b)
    for b in (0, 1):
        wait_o(b)


def kernel(tokens, wte_weight):
    idx1d = tokens.astype(jnp.int32).reshape(TOTAL)
    out4 = _sc_gather(wte_weight, idx1d)
    o5 = out4.reshape(SEQ, ER, NW, 8, 128)
    return o5.transpose(2, 4, 0, 1, 3).reshape(BATCH, SEQ, EMBED)


# speedup vs baseline: 1.6344x; 1.6228x over previous
import functools

import jax
import jax.numpy as jnp
from jax import lax
from jax.experimental import pallas as pl
from jax.experimental.pallas import tpu as pltpu
from jax.experimental.pallas import tpu_sc as plsc

EMBED = 64
LANES = 16
NC, NS = 2, 16
NW = NC * NS
BATCH = 4096
SEQ = 200
TOTAL = BATCH * SEQ
BPW = BATCH // NW
PER_W = BPW * SEQ
ER = EMBED // 8
TILE = 8 * 128


@functools.partial(
    pl.kernel,
    mesh=plsc.VectorSubcoreMesh(core_axis_name="c", subcore_axis_name="s"),
    out_type=jax.ShapeDtypeStruct((SEQ, ER, NW, TILE), jnp.float32),
    scratch_types=[
        pltpu.VMEM((PER_W,), jnp.int32),
        pltpu.VMEM((PER_W,), jnp.int32),
        pltpu.VMEM((2, BPW, EMBED), jnp.float32),
        pltpu.VMEM((2, ER, TILE), jnp.float32),
        pltpu.SemaphoreType.DMA,
        pltpu.SemaphoreType.DMA,
        pltpu.SemaphoreType.DMA,
        pltpu.SemaphoreType.DMA,
    ],
    compiler_params=pltpu.CompilerParams(
        use_tc_tiling_on_sc=False, needs_layout_passes=False),
)
def _sc_gather(table_hbm, idx_hbm, out_hbm, idx_v, idxt_v, rows_v, tiles_v,
               g0, g1, o0, o1):
    gsem = (g0, g1)
    osem = (o0, o1)
    wid = lax.axis_index("s") * NC + lax.axis_index("c")
    base = wid * PER_W
    pltpu.sync_copy(idx_hbm.at[pl.ds(base, PER_W)], idx_v)

    iota = lax.iota(jnp.int32, LANES)
    pre_s = [(16 * g + iota) * SEQ for g in range(BPW // LANES)]

    @plsc.parallel_loop(0, SEQ, unroll=2)
    def _build_idxt(s):
        for g in range(BPW // LANES):
            val = plsc.load_gather(idx_v, [pre_s[g] + s])
            idxt_v[pl.ds(s * BPW + 16 * g, LANES)] = val

    def fire_g(s, b):
        pltpu.async_copy(
            table_hbm.at[idxt_v.at[pl.ds(s * BPW, BPW)]],
            rows_v.at[b],
            gsem[b],
        )

    def wait_g(b):
        pltpu.make_async_copy(
            table_hbm.at[pl.ds(0, BPW), :],
            rows_v.at[b],
            gsem[b],
        ).wait()

    row_vec = [16 * g + iota for g in range(BPW // LANES)]

    def transpose(b):
        tiles2d = tiles_v.at[b]
        @plsc.parallel_loop(0, EMBED, unroll=8)
        def _t(e):
            ecol = (e + iota) & 63
            er_v = ecol >> 3
            cb_v = (ecol & 7) << 7
            for g in range(BPW // LANES):
                val = plsc.load_gather(rows_v.at[b], [row_vec[g], ecol])
                plsc.store_scatter(tiles2d, [er_v, cb_v + row_vec[g]], val)

    def fire_o(s, b):
        pltpu.async_copy(
            tiles_v.at[b],
            out_hbm.at[s, :, wid],
            osem[b],
        )

    def wait_o(b):
        pltpu.make_async_copy(
            tiles_v.at[b],
            out_hbm.at[0, :, 0],
            osem[b],
        ).wait()

    fire_g(0, 0)
    fire_g(1, 1)
    for u in (0, 1):
        wait_g(u)
        transpose(u)
        fire_o(u, u)
        fire_g(u + 2, u)

    def unit(u, b):
        wait_g(b)
        wait_o(b)
        transpose(b)
        fire_o(u, b)
        fire_g(u + 2, b)

    def pair(p, carry):
        u = 2 + 2 * p
        unit(u, 0)
        unit(u + 1, 1)
        return carry

    lax.fori_loop(0, (SEQ - 4) // 2, pair, 0)
    for u in (SEQ - 2, SEQ - 1):
        b = u % 2
        wait_g(b)
        wait_o(b)
        transpose(b)
        fire_o(u, b)
    for b in (0, 1):
        wait_o(b)


def kernel(tokens, wte_weight):
    idx1d = tokens.astype(jnp.int32).reshape(TOTAL)
    out4 = _sc_gather(wte_weight, idx1d)
    o5 = out4.reshape(SEQ, ER, NW, 8, 128)
    return o5.transpose(2, 4, 0, 1, 3).reshape(BATCH, SEQ, EMBED)


# speedup vs baseline: 3.0471x; 1.8643x over previous
import functools

import jax
import jax.numpy as jnp
from jax import lax
from jax.experimental import pallas as pl
from jax.experimental.pallas import tpu as pltpu
from jax.experimental.pallas import tpu_sc as plsc

EMBED = 64
LANES = 16
NC, NS = 2, 16
NW = NC * NS
BATCH = 4096
SEQ = 200
TOTAL = BATCH * SEQ
BPW = BATCH // NW
PER_W = BPW * SEQ
ER = EMBED // 8
TILE = 8 * 128


@functools.partial(
    pl.kernel,
    mesh=plsc.VectorSubcoreMesh(core_axis_name="c", subcore_axis_name="s"),
    out_type=jax.ShapeDtypeStruct((SEQ, ER, NW, TILE), jnp.float32),
    scratch_types=[
        pltpu.VMEM((PER_W,), jnp.int32),
        pltpu.VMEM((PER_W,), jnp.int32),
        pltpu.VMEM((2, BPW, EMBED), jnp.float32),
        pltpu.VMEM((2, ER, TILE), jnp.float32),
        pltpu.SemaphoreType.DMA,
        pltpu.SemaphoreType.DMA,
        pltpu.SemaphoreType.DMA,
        pltpu.SemaphoreType.DMA,
    ],
    compiler_params=pltpu.CompilerParams(
        use_tc_tiling_on_sc=False, needs_layout_passes=False),
)
def _sc_gather(table_hbm, idx_hbm, out_hbm, idx_v, idxt_v, rows_v, tiles_v,
               g0, g1, o0, o1):
    gsem = (g0, g1)
    osem = (o0, o1)
    wid = lax.axis_index("s") * NC + lax.axis_index("c")
    base = wid * PER_W
    pltpu.sync_copy(idx_hbm.at[pl.ds(base, PER_W)], idx_v)

    iota = lax.iota(jnp.int32, LANES)
    pre_s = [(16 * g + iota) * SEQ for g in range(BPW // LANES)]

    @plsc.parallel_loop(0, SEQ, unroll=2)
    def _build_idxt(s):
        for g in range(BPW // LANES):
            val = plsc.load_gather(idx_v, [pre_s[g] + s])
            idxt_v[pl.ds(s * BPW + 16 * g, LANES)] = val

    def fire_g(s, b):
        pltpu.async_copy(
            table_hbm.at[idxt_v.at[pl.ds(s * BPW, BPW)]],
            rows_v.at[b],
            gsem[b],
        )

    def wait_g(b):
        pltpu.make_async_copy(
            table_hbm.at[pl.ds(0, BPW), :],
            rows_v.at[b],
            gsem[b],
        ).wait()

    row_vec = [16 * g + iota for g in range(BPW // LANES)]

    def transpose(b):
        tiles2d = tiles_v.at[b]
        @plsc.parallel_loop(0, EMBED, unroll=8)
        def _t(e):
            ecol = (e + iota) & 63
            er_v = ecol >> 3
            cb_v = (ecol & 7) << 7
            for g in range(BPW // LANES):
                val = plsc.load_gather(rows_v.at[b], [row_vec[g], ecol])
                plsc.store_scatter(tiles2d, [er_v, cb_v + row_vec[g]], val)

    def fire_o(s, b):
        pltpu.async_copy(
            tiles_v.at[b],
            out_hbm.at[s, :, wid],
            osem[b],
        )

    def wait_o(b):
        pltpu.make_async_copy(
            tiles_v.at[b],
            out_hbm.at[0, :, 0],
            osem[b],
        ).wait()

    fire_g(0, 0)
    fire_g(1, 1)
    for u in (0, 1):
        wait_g(u)
        transpose(u)
        fire_o(u, u)
        fire_g(u + 2, u)

    def unit(u, b):
        wait_g(b)
        wait_o(b)
        transpose(b)
        fire_o(u, b)
        fire_g(u + 2, b)

    def pair(p, carry):
        u = 2 + 2 * p
        unit(u, 0)
        unit(u + 1, 1)
        return carry

    lax.fori_loop(0, (SEQ - 4) // 2, pair, 0)
    for u in (SEQ - 2, SEQ - 1):
        b = u % 2
        wait_g(b)
        wait_o(b)
        transpose(b)
        fire_o(u, b)
    for b in (0, 1):
        wait_o(b)


VOCAB = 1000000
NCOL = VOCAB // 128
FULL_PER_W = NCOL // NW
REM = NCOL - FULL_PER_W * NW
TAIL = VOCAB - NCOL * 128


@functools.partial(
    pl.kernel,
    mesh=plsc.VectorSubcoreMesh(core_axis_name="c", subcore_axis_name="s"),
    out_type=jax.ShapeDtypeStruct((VOCAB // 2, 128), jnp.float32),
    scratch_types=[
        pltpu.VMEM((2, EMBED, 128), jnp.float32),
        pltpu.VMEM((2, EMBED, 128), jnp.float32),
        pltpu.VMEM((EMBED, TAIL), jnp.float32),
        pltpu.VMEM((TAIL // 2, 128), jnp.float32),
        pltpu.SemaphoreType.DMA,
        pltpu.SemaphoreType.DMA,
        pltpu.SemaphoreType.DMA,
        pltpu.SemaphoreType.DMA,
    ],
    compiler_params=pltpu.CompilerParams(needs_layout_passes=False),
)
def _sc_relayout(tw_hbm, out_hbm, col_v, row_v, tcol_v, trow_v,
                 g0, g1, o0, o1):
    gsem = (g0, g1)
    osem = (o0, o1)
    wid = lax.axis_index("s") * NC + lax.axis_index("c")
    iota = lax.iota(jnp.int32, LANES)
    row16 = [16 * g + iota for g in range(128 // LANES)]

    def colof(k):
        return wid + NW * k

    def fire_in(k, b):
        pltpu.async_copy(
            tw_hbm.at[:, pl.ds(colof(k) * 128, 128)], col_v.at[b], gsem[b])

    def wait_in(b):
        pltpu.make_async_copy(
            tw_hbm.at[:, pl.ds(0, 128)], col_v.at[b], gsem[b]).wait()

    def fire_out(k, b):
        pltpu.async_copy(
            row_v.at[b], out_hbm.at[pl.ds(colof(k) * 64, 64), :], osem[b])

    def wait_out(b):
        pltpu.make_async_copy(
            row_v.at[b], out_hbm.at[pl.ds(0, 64), :], osem[b]).wait()

    def transpose_col(b):
        rdst = row_v.at[b]
        @plsc.parallel_loop(0, 128, unroll=8)
        def _t(v):
            vrow = (v + iota) & 127
            vhalf = vrow >> 1
            vc0 = (vrow & 1) << 6
            for g in range(EMBED // LANES):
                val = plsc.load_gather(col_v.at[b], [row16[g], vrow])
                plsc.store_scatter(rdst, [vhalf, vc0 + row16[g]], val)

    fire_in(0, 0)
    fire_in(1, 1)
    for k in (0, 1):
        wait_in(k)
        transpose_col(k)
        fire_out(k, k)
        fire_in(k + 2, k)

    def unit(k, b):
        wait_in(b)
        wait_out(b)
        transpose_col(b)
        fire_out(k, b)
        fire_in(k + 2, b)

    def pairk(p, carry):
        k = 2 + 2 * p
        unit(k, 0)
        unit(k + 1, 1)
        return carry

    lax.fori_loop(0, (FULL_PER_W - 4) // 2, pairk, 0)
    for k in (FULL_PER_W - 2, FULL_PER_W - 1):
        b = k % 2
        wait_in(b)
        wait_out(b)
        transpose_col(b)
        fire_out(k, b)
    wait_out(0)
    wait_out(1)
    @pl.when(wid < REM)
    def _extra():
        c = NCOL - REM + wid
        pltpu.sync_copy(tw_hbm.at[:, pl.ds(c * 128, 128)], col_v.at[0])
        transpose_col(0)
        pltpu.sync_copy(row_v.at[0], out_hbm.at[pl.ds(c * 64, 64), :])
    @pl.when(wid == REM)
    def _tail():
        pltpu.sync_copy(tw_hbm.at[:, pl.ds(NCOL * 128, TAIL)], tcol_v)
        @plsc.parallel_loop(0, TAIL, unroll=8)
        def _tt(v):
            vrow = (v + iota) & (TAIL - 1)
            vhalf = vrow >> 1
            vc0 = (vrow & 1) << 6
            for g in range(EMBED // LANES):
                val = plsc.load_gather(tcol_v, [row16[g], vrow])
                plsc.store_scatter(trow_v, [vhalf, vc0 + row16[g]], val)
        pltpu.sync_copy(trow_v, out_hbm.at[pl.ds(NCOL * 64, TAIL // 2), :])


def kernel(tokens, wte_weight):
    idx1d = tokens.astype(jnp.int32).reshape(TOTAL)
    table_rm = _sc_relayout(wte_weight.T).reshape(VOCAB, EMBED)
    out4 = _sc_gather(table_rm, idx1d)
    o5 = out4.reshape(SEQ, ER, NW, 8, 128)
    return o5.transpose(2, 4, 0, 1, 3).reshape(BATCH, SEQ, EMBED)
